# baseline (speedup 1.0000x reference)
import functools

import jax
import jax.numpy as jnp
from jax import lax
from jax.experimental import pallas as pl
from jax.experimental.pallas import tpu as pltpu
from jax.experimental.pallas import tpu_sc as plsc


def kernel(x, pos_table):
    seq_len = x.shape[1]
    emb_dim = pos_table.shape[1]

    info = plsc.get_sparse_core_info()
    num_cores, num_subcores = info.num_cores, info.num_subcores
    num_workers = num_cores * num_subcores
    assert seq_len % num_workers == 0
    rows_per_worker = seq_len // num_workers

    mesh = plsc.VectorSubcoreMesh(core_axis_name="c", subcore_axis_name="s")

    @functools.partial(
        pl.kernel,
        mesh=mesh,
        out_type=jax.ShapeDtypeStruct((seq_len, emb_dim), jnp.float32),
    )
    def copy_rows(table_hbm, out_hbm):
        wid = lax.axis_index("s") * num_cores + lax.axis_index("c")
        base = wid * rows_per_worker
        pltpu.sync_copy(
            table_hbm.at[pl.ds(base, rows_per_worker)],
            out_hbm.at[pl.ds(base, rows_per_worker)],
        )

    return copy_rows(pos_table)[None]


# speedup vs baseline: 23.1764x; 23.1764x over previous
import functools

import jax
import jax.numpy as jnp
from jax import lax
from jax.experimental import pallas as pl
from jax.experimental.pallas import tpu as pltpu
from jax.experimental.pallas import tpu_sc as plsc

_CHUNK_ROWS = 32


def kernel(x, pos_table):
    seq_len = x.shape[1]
    emb_dim = pos_table.shape[1]

    info = plsc.get_sparse_core_info()
    num_cores, num_subcores = info.num_cores, info.num_subcores
    num_workers = num_cores * num_subcores
    assert seq_len % (num_workers * _CHUNK_ROWS) == 0
    rows_per_worker = seq_len // num_workers
    nchunks = rows_per_worker // _CHUNK_ROWS

    mesh = plsc.VectorSubcoreMesh(core_axis_name="c", subcore_axis_name="s")

    @functools.partial(
        pl.kernel,
        mesh=mesh,
        out_type=jax.ShapeDtypeStruct((seq_len, emb_dim), jnp.float32),
        scratch_types=[
            pltpu.VMEM((_CHUNK_ROWS, emb_dim), jnp.float32),
            pltpu.VMEM((_CHUNK_ROWS, emb_dim), jnp.float32),
            pltpu.SemaphoreType.DMA,
            pltpu.SemaphoreType.DMA,
        ],
    )
    def copy_rows(table_hbm, out_hbm, buf0, buf1, lsem, ssem):
        wid = lax.axis_index("s") * num_cores + lax.axis_index("c")
        base = wid * rows_per_worker
        bufs = (buf0, buf1)

        def load(i):
            return pltpu.make_async_copy(
                table_hbm.at[pl.ds(base + i * _CHUNK_ROWS, _CHUNK_ROWS)],
                bufs[i % 2], lsem)

        def store(i):
            return pltpu.make_async_copy(
                bufs[i % 2],
                out_hbm.at[pl.ds(base + i * _CHUNK_ROWS, _CHUNK_ROWS)], ssem)

        load(0).start()
        load(0).wait()
        store(0).start()
        if nchunks > 1:
            load(1).start()
        for i in range(1, nchunks):
            load(i).wait()
            store(i - 1).wait()
            store(i).start()
            if i + 1 < nchunks:
                load(i + 1).start()
        store(nchunks - 1).wait()

    return copy_rows(pos_table)[None]


# speedup vs baseline: 24.7771x; 1.0691x over previous
import functools

import jax
import jax.numpy as jnp
from jax import lax
from jax.experimental import pallas as pl
from jax.experimental.pallas import tpu as pltpu
from jax.experimental.pallas import tpu_sc as plsc

_CHUNK_ROWS = 32
_NBUF = 3


def kernel(x, pos_table):
    seq_len = x.shape[1]
    emb_dim = pos_table.shape[1]

    info = plsc.get_sparse_core_info()
    num_cores, num_subcores = info.num_cores, info.num_subcores
    num_workers = num_cores * num_subcores
    assert seq_len % (num_workers * _CHUNK_ROWS) == 0
    rows_per_worker = seq_len // num_workers
    nchunks = rows_per_worker // _CHUNK_ROWS

    mesh = plsc.VectorSubcoreMesh(core_axis_name="c", subcore_axis_name="s")

    @functools.partial(
        pl.kernel,
        mesh=mesh,
        out_type=jax.ShapeDtypeStruct((seq_len, emb_dim), jnp.float32),
        scratch_types=(
            [pltpu.VMEM((_CHUNK_ROWS, emb_dim), jnp.float32) for _ in range(_NBUF)]
            + [pltpu.SemaphoreType.DMA, pltpu.SemaphoreType.DMA]
        ),
    )
    def copy_rows(table_hbm, out_hbm, *rest):
        bufs, (lsem, ssem) = rest[:_NBUF], rest[_NBUF:]
        wid = lax.axis_index("s") * num_cores + lax.axis_index("c")
        base = wid * rows_per_worker

        def load(i):
            return pltpu.make_async_copy(
                table_hbm.at[pl.ds(base + i * _CHUNK_ROWS, _CHUNK_ROWS)],
                bufs[i % _NBUF], lsem)

        def store(i):
            return pltpu.make_async_copy(
                bufs[i % _NBUF],
                out_hbm.at[pl.ds(base + i * _CHUNK_ROWS, _CHUNK_ROWS)], ssem)

        store_waited = [False] * nchunks
        for j in range(min(2, nchunks)):
            load(j).start()
        for i in range(nchunks):
            load(i).wait()
            store(i).start()
            if i + 2 < nchunks:
                if i - 1 >= 0:
                    store(i - 1).wait()
                    store_waited[i - 1] = True
                load(i + 2).start()
        for i in range(nchunks):
            if not store_waited[i]:
                store(i).wait()

    return copy_rows(pos_table)[None]
